# Initial kernel scaffold; baseline (speedup 1.0000x reference)
import jax
import jax.numpy as jnp
from jax.experimental import pallas as pl


def kernel(obs, k, z, mW1, mb1, mW2, mb2, mW3, mb3, sW1, sb1, sW2, sb2, sW3, sb3):
    raise NotImplementedError("write your pallas kernel here")


# speedup vs baseline: 2.3864x; 2.3864x over previous
import functools

import jax
import jax.numpy as jnp
from jax import lax
from jax.experimental import pallas as pl
from jax.experimental.pallas import tpu as pltpu
from jax.experimental.pallas import tpu_sc as plsc

N = 32768
NUM_MIX = 8
NCH = 2 * NUM_MIX
NC = 2
NS = 16
LANES = 16
NW = NC * NS
CHUNK = N // NW
NSLICE = CHUNK // LANES


def _sc_body(obs_hbm, z_hbm, ct2_hbm, a2_hbm, w2_hbm, b2_hbm, w3_hbm, b3_hbm,
             mean_hbm, std_hbm,
             obs_v, z_v, ct2_v, a2_v, w2_v, b2_v, w3_v, b3_v, om_v, os_v):
    wid = lax.axis_index("c") * NS + lax.axis_index("s")
    base = wid * CHUNK

    pltpu.sync_copy(obs_hbm.at[pl.ds(base, CHUNK)], obs_v)
    pltpu.sync_copy(z_hbm.at[pl.ds(base, CHUNK)], z_v)
    pltpu.sync_copy(ct2_hbm, ct2_v)
    pltpu.sync_copy(a2_hbm, a2_v)
    pltpu.sync_copy(w2_hbm, w2_v)
    pltpu.sync_copy(b2_hbm, b2_v)
    pltpu.sync_copy(w3_hbm, w3_v)
    pltpu.sync_copy(b3_hbm, b3_v)

    def row(ref, r):
        return ref[pl.ds(r * LANES, LANES)]

    def slice_body(s, carry):
        o = s * LANES
        obs16 = obs_v[pl.ds(o, LANES)]
        z16 = z_v[pl.ds(o, LANES)] * NCH

        h1 = []
        for j in range(NCH):
            cz = plsc.load_gather(ct2_v, [z16 + j])
            t = jnp.exp(obs16 * row(a2_v, j) + cz)
            h1.append((t - 1.0) / (t + 1.0))

        macc = row(b3_v, 0)
        sacc = row(b3_v, 1)
        for i in range(NCH):
            acc = row(b2_v, i)
            off = (i // NUM_MIX) * NUM_MIX
            for j in range(NUM_MIX):
                acc = acc + row(w2_v, i * NUM_MIX + j) * h1[off + j]
            t = jnp.exp(acc)
            h2 = (t - 1.0) / (t + 1.0)
            if i < NUM_MIX:
                macc = macc + row(w3_v, i) * h2
            else:
                sacc = sacc + row(w3_v, i) * h2

        om_v[pl.ds(o, LANES)] = macc
        os_v[pl.ds(o, LANES)] = jnp.exp(sacc)
        return carry

    lax.fori_loop(0, NSLICE, slice_body, None)

    pltpu.sync_copy(om_v, mean_hbm.at[pl.ds(base, CHUNK)])
    pltpu.sync_copy(os_v, std_hbm.at[pl.ds(base, CHUNK)])


def _scratch_types():
    return [
        pltpu.VMEM((CHUNK,), jnp.float32),
        pltpu.VMEM((CHUNK,), jnp.int32),
        pltpu.VMEM((NUM_MIX * NCH,), jnp.float32),
        pltpu.VMEM((NCH * LANES,), jnp.float32),
        pltpu.VMEM((NCH * NUM_MIX * LANES,), jnp.float32),
        pltpu.VMEM((NCH * LANES,), jnp.float32),
        pltpu.VMEM((NCH * LANES,), jnp.float32),
        pltpu.VMEM((2 * LANES,), jnp.float32),
        pltpu.VMEM((CHUNK,), jnp.float32),
        pltpu.VMEM((CHUNK,), jnp.float32),
    ]


@functools.cache
def _sc_call():
    return functools.partial(
        pl.kernel,
        out_type=(
            jax.ShapeDtypeStruct((N,), jnp.float32),
            jax.ShapeDtypeStruct((N,), jnp.float32),
        ),
        mesh=plsc.VectorSubcoreMesh(
            core_axis_name="c", subcore_axis_name="s",
            num_cores=NC, num_subcores=NS,
        ),
        scratch_types=_scratch_types(),
        compiler_params=pltpu.CompilerParams(needs_layout_passes=False),
    )(_sc_body)


def kernel(obs, k, z, mW1, mb1, mW2, mb2, mW3, mb3,
           sW1, sb1, sW2, sb2, sW3, sb3):
    del k
    a2 = 2.0 * jnp.concatenate([mW1[:, 0], sW1[:, 0]])
    ct2 = (2.0 * jnp.concatenate(
        [mW1[:, 1:].T + mb1[None, :], sW1[:, 1:].T + sb1[None, :]], axis=1
    )).reshape(-1)
    w2 = (2.0 * jnp.concatenate([mW2, sW2], axis=0)).reshape(-1)
    b2 = 2.0 * jnp.concatenate([mb2, sb2])
    w3 = jnp.concatenate([mW3[0], sW3[0]])
    b3 = jnp.concatenate([mb3, sb3])
    mean, std = _sc_call()(
        obs, z.astype(jnp.int32), ct2,
        jnp.repeat(a2, LANES), jnp.repeat(w2, LANES),
        jnp.repeat(b2, LANES), jnp.repeat(w3, LANES),
        jnp.repeat(b3, LANES),
    )
    return mean, std
